# baseline (speedup 1.0000x reference)
import jax
import jax.numpy as jnp
from jax.experimental import pallas as pl
from jax.experimental.pallas import tpu as pltpu

_ROWS = 100000
_D = 128
_BS = 2000
_GRID = _ROWS // _BS


def _body(xs_ref, x_ref, out_ref):
    out_ref[...] = xs_ref[...]

    @pl.when(pl.program_id(0) == 0)
    def _():
        out_ref[0:1, :] = x_ref[...]


@jax.jit
def kernel(xs, x):
    return pl.pallas_call(
        _body,
        grid=(_GRID,),
        out_shape=jax.ShapeDtypeStruct((_ROWS, _D), jnp.float32),
        in_specs=[
            pl.BlockSpec((_BS, _D), lambda i: (i, 0)),
            pl.BlockSpec((1, _D), lambda i: (0, 0)),
        ],
        out_specs=pl.BlockSpec((_BS, _D), lambda i: (i, 0)),
        compiler_params=pltpu.CompilerParams(
            dimension_semantics=("arbitrary",),
        ),
    )(xs, x)


# speedup vs baseline: 1.5579x; 1.5579x over previous
import jax
import jax.numpy as jnp
from jax.experimental import pallas as pl
from jax.experimental.pallas import tpu as pltpu

_ROWS = 100000
_D = 128
_BS = 10000
_GRID = _ROWS // _BS


def _body(xs_ref, x_ref, out_ref):
    out_ref[...] = xs_ref[...]

    @pl.when(pl.program_id(0) == 0)
    def _():
        out_ref[0:1, :] = x_ref[...]


@jax.jit
def kernel(xs, x):
    return pl.pallas_call(
        _body,
        grid=(_GRID,),
        out_shape=jax.ShapeDtypeStruct((_ROWS, _D), jnp.float32),
        in_specs=[
            pl.BlockSpec((_BS, _D), lambda i: (i, 0)),
            pl.BlockSpec((1, _D), lambda i: (0, 0)),
        ],
        out_specs=pl.BlockSpec((_BS, _D), lambda i: (i, 0)),
        compiler_params=pltpu.CompilerParams(
            dimension_semantics=("arbitrary",),
        ),
    )(xs, x)


# speedup vs baseline: 1.6318x; 1.0474x over previous
import jax
import jax.numpy as jnp
from jax.experimental import pallas as pl
from jax.experimental.pallas import tpu as pltpu

_ROWS = 100000
_D = 128
_BS = 20000
_GRID = _ROWS // _BS


def _body(xs_ref, x_ref, out_ref):
    out_ref[...] = xs_ref[...]

    @pl.when(pl.program_id(0) == 0)
    def _():
        out_ref[0:1, :] = x_ref[...]


@jax.jit
def kernel(xs, x):
    return pl.pallas_call(
        _body,
        grid=(_GRID,),
        out_shape=jax.ShapeDtypeStruct((_ROWS, _D), jnp.float32),
        in_specs=[
            pl.BlockSpec((_BS, _D), lambda i: (i, 0)),
            pl.BlockSpec((1, _D), lambda i: (0, 0)),
        ],
        out_specs=pl.BlockSpec((_BS, _D), lambda i: (i, 0)),
        compiler_params=pltpu.CompilerParams(
            dimension_semantics=("arbitrary",),
        ),
    )(xs, x)
